# Initial kernel scaffold; baseline (speedup 1.0000x reference)
import jax
import jax.numpy as jnp
from jax.experimental import pallas as pl


def kernel(x, edge_index, W1, b1, W2, b2):
    raise NotImplementedError("write your pallas kernel here")


# speedup vs baseline: 9.7852x; 9.7852x over previous
import functools

import jax
import jax.numpy as jnp
from jax import lax
from jax.experimental import pallas as pl
from jax.experimental.pallas import tpu as pltpu
from jax.experimental.pallas import tpu_sc as plsc

NC = 2
NS = 16
NW = NC * NS
CHUNK = 128
R = 5
RANGE = 10016
ACC_ROWS = 10048
STRIPE = ACC_ROWS // NS
TRASH = RANGE
DEG_ROWS = 51200
DEG_STRIPE = DEG_ROWS // NS


def _mesh():
    return plsc.VectorSubcoreMesh(core_axis_name="c", subcore_axis_name="s")


def _sc_params():
    return pltpu.CompilerParams(use_tc_tiling_on_sc=False,
                                needs_layout_passes=False)


def _fill_const(buf, rows, width, value):
    def body(i, _):
        for w0 in range(width // 16):
            buf[i, pl.ds(w0 * 16, 16)] = jnp.full((16,), value, jnp.float32)
        return 0
    lax.fori_loop(0, rows, body, 0)


def _bucket_kernel(total_chunks, cap):
    cpt = total_chunks // NW

    @functools.partial(
        pl.kernel,
        out_type=[
            jax.ShapeDtypeStruct((NW, R, cap), jnp.int32),
            jax.ShapeDtypeStruct((NW, R, cap), jnp.int32),
            jax.ShapeDtypeStruct((NW, 8, 16), jnp.int32),
        ],
        mesh=_mesh(),
        compiler_params=_sc_params(),
        scratch_types=[
            pltpu.VMEM((cpt, CHUNK), jnp.int32),
            pltpu.VMEM((cpt, CHUNK), jnp.int32),
            pltpu.VMEM((cap,), jnp.int32),
            pltpu.VMEM((cap,), jnp.int32),
            pltpu.VMEM((8, 16), jnp.int32),
        ],
    )
    def k(src_hbm, dst_hbm,
          bsrc_hbm, bdst_hbm, nch_hbm,
          si_v, di_v, cs_v, cd_v, cnts_v):
        c = lax.axis_index("c")
        s = lax.axis_index("s")
        g = c * NS + s
        pltpu.sync_copy(src_hbm.at[pl.ds(g * cpt, cpt)], si_v)
        pltpu.sync_copy(dst_hbm.at[pl.ds(g * cpt, cpt)], di_v)

        for r in range(R):
            lo = r * RANGE

            def scan_body(it, cnt):
                j = it // 8
                q = it % 8
                vd = di_v[j, pl.ds(q * 16, 16)]
                vs = si_v[j, pl.ds(q * 16, 16)]
                mask = (vd >= lo) & (vd < lo + RANGE)
                loc = vd - lo
                pos = plsc.cumsum(mask.astype(jnp.int32))
                idx = pos - 1 + cnt
                plsc.store_scatter(cs_v, [idx], vs, mask=mask)
                plsc.store_scatter(cd_v, [idx], loc, mask=mask)
                return cnt + pos[15]
            cnt = lax.fori_loop(0, cpt * 8, scan_body, jnp.int32(0))

            for kk in range(9):
                cs_v[pl.ds(cnt + kk * 16, 16)] = jnp.zeros((16,), jnp.int32)
                cd_v[pl.ds(cnt + kk * 16, 16)] = jnp.full((16,), TRASH,
                                                          jnp.int32)
            nc = (cnt + CHUNK - 1) // CHUNK
            cnts_v[r, :] = jnp.full((16,), 1, jnp.int32) * nc
            pltpu.sync_copy(cs_v, bsrc_hbm.at[g, r])
            pltpu.sync_copy(cd_v, bdst_hbm.at[g, r])
        for r in range(R, 8):
            cnts_v[r, :] = jnp.zeros((16,), jnp.int32)
        pltpu.sync_copy(cnts_v, nch_hbm.at[g])

    return k


def _scatter_kernel(cap, width, buckets_per_tile):
    @functools.partial(
        pl.kernel,
        out_type=jax.ShapeDtypeStruct((NC, R * ACC_ROWS, width), jnp.float32),
        mesh=_mesh(),
        compiler_params=_sc_params(),
        scratch_types=[
            pltpu.VMEM((cap,), jnp.int32),
            pltpu.VMEM((cap,), jnp.int32),
            pltpu.VMEM((CHUNK,), jnp.int32),
            pltpu.VMEM((CHUNK, width), jnp.float32),
            pltpu.VMEM((16,), jnp.int32),
            pltpu.VMEM((STRIPE, width), jnp.float32),
            pltpu.VMEM_SHARED((ACC_ROWS, width), jnp.float32),
        ],
    )
    def k(bsrc_hbm, bdst_hbm, nch_hbm, ta_hbm, tb_hbm, out_hbm,
          sb_v, db_v, dchunk_v, gbuf_v, ncv_v, zeros_v, acc_s):
        c = lax.axis_index("c")
        s = lax.axis_index("s")
        _fill_const(zeros_v, STRIPE, width, 0.0)
        pltpu.sync_copy(zeros_v, acc_s.at[pl.ds(s * STRIPE, STRIPE)])
        plsc.subcore_barrier()

        def run_round(table, r):
            for kk in range(buckets_per_tile):
                if buckets_per_tile == 2:
                    bt = 2 * s + kk
                else:
                    bt = c * NS + s
                pltpu.sync_copy(nch_hbm.at[bt, r], ncv_v)
                nc = ncv_v[...][0]
                pltpu.sync_copy(bsrc_hbm.at[bt, r], sb_v)
                pltpu.sync_copy(bdst_hbm.at[bt, r], db_v)

                def chunk_body(j, _):
                    for q in range(CHUNK // 16):
                        dchunk_v[pl.ds(q * 16, 16)] = (
                            db_v[pl.ds(j * CHUNK + q * 16, 16)])
                    pltpu.sync_copy(
                        table.at[sb_v.at[pl.ds(j * CHUNK, CHUNK)]], gbuf_v)
                    pltpu.sync_copy(gbuf_v, acc_s.at[dchunk_v], add=True)
                    return 0
                lax.fori_loop(0, nc, chunk_body, 0)

        for r in range(R):
            if buckets_per_tile == 2:
                @pl.when(c == 0)
                def _():
                    run_round(ta_hbm, r)

                @pl.when(c == 1)
                def _():
                    run_round(tb_hbm, r)
            else:
                run_round(ta_hbm, r)
            plsc.subcore_barrier()
            pltpu.sync_copy(
                acc_s.at[pl.ds(s * STRIPE, STRIPE)],
                out_hbm.at[c, pl.ds(r * ACC_ROWS + s * STRIPE, STRIPE)])
            if r < R - 1:
                pltpu.sync_copy(zeros_v,
                                acc_s.at[pl.ds(s * STRIPE, STRIPE)])
            plsc.subcore_barrier()

    return k


def _tc1(x, W1, d0, d1, rb):
    n, d_in = x.shape
    d_out = W1.shape[1]

    def body(x_ref, w_ref, d0_ref, d1_ref, o_ref):
        dinv = lax.rsqrt(d0_ref[...] + d1_ref[...] + 1.0)
        h = jnp.dot(x_ref[...], w_ref[...], preferred_element_type=jnp.float32)
        o_ref[...] = h * dinv

    return pl.pallas_call(
        body,
        grid=(n // rb,),
        in_specs=[
            pl.BlockSpec((rb, d_in), lambda i: (i, 0)),
            pl.BlockSpec((d_in, d_out), lambda i: (0, 0)),
            pl.BlockSpec((rb, 1), lambda i: (i, 0)),
            pl.BlockSpec((rb, 1), lambda i: (i, 0)),
        ],
        out_specs=pl.BlockSpec((rb, d_out), lambda i: (i, 0)),
        out_shape=jax.ShapeDtypeStruct((n, d_out), jnp.float32),
    )(x, W1, d0, d1)


def _tc2(a0, a1, h1p, d0, d1, W2, b1, rb):
    n, half = a0.shape
    d2 = W2.shape[1]

    def body(a0_ref, a1_ref, h_ref, d0_ref, d1_ref, w_ref, b_ref, o_ref):
        dinv = lax.rsqrt(d0_ref[...] + d1_ref[...] + 1.0)
        u = jnp.concatenate([a0_ref[...], a1_ref[...]], axis=1) + h_ref[...]
        z = jnp.maximum(u * dinv + b_ref[...], 0.0)
        h2 = jnp.dot(z, w_ref[...], preferred_element_type=jnp.float32)
        o_ref[...] = h2 * dinv

    return pl.pallas_call(
        body,
        grid=(n // rb,),
        in_specs=[
            pl.BlockSpec((rb, half), lambda i: (i, 0)),
            pl.BlockSpec((rb, half), lambda i: (i, 0)),
            pl.BlockSpec((rb, 2 * half), lambda i: (i, 0)),
            pl.BlockSpec((rb, 1), lambda i: (i, 0)),
            pl.BlockSpec((rb, 1), lambda i: (i, 0)),
            pl.BlockSpec((2 * half, d2), lambda i: (0, 0)),
            pl.BlockSpec((1, 2 * half), lambda i: (0, 0)),
        ],
        out_specs=pl.BlockSpec((rb, d2), lambda i: (i, 0)),
        out_shape=jax.ShapeDtypeStruct((n, d2), jnp.float32),
    )(a0, a1, h1p, d0, d1, W2, b1)


def _tc3(p0, p1, h2p, d0, d1, b2, rb):
    n, d2 = h2p.shape

    def body(p0_ref, p1_ref, h_ref, d0_ref, d1_ref, b_ref, o_ref):
        dinv = lax.rsqrt(d0_ref[...] + d1_ref[...] + 1.0)
        u = p0_ref[...] + p1_ref[...] + h_ref[...]
        o_ref[...] = jnp.maximum(u * dinv + b_ref[...], 0.0)

    return pl.pallas_call(
        body,
        grid=(n // rb,),
        in_specs=[
            pl.BlockSpec((rb, d2), lambda i: (i, 0)),
            pl.BlockSpec((rb, d2), lambda i: (i, 0)),
            pl.BlockSpec((rb, d2), lambda i: (i, 0)),
            pl.BlockSpec((rb, 1), lambda i: (i, 0)),
            pl.BlockSpec((rb, 1), lambda i: (i, 0)),
            pl.BlockSpec((1, d2), lambda i: (0, 0)),
        ],
        out_specs=pl.BlockSpec((rb, d2), lambda i: (i, 0)),
        out_shape=jax.ShapeDtypeStruct((n, d2), jnp.float32),
    )(p0, p1, h2p, d0, d1, b2)


def _reassemble(y, n):
    parts = [y[:, r * ACC_ROWS: r * ACC_ROWS + RANGE, :] for r in range(R)]
    full = jnp.concatenate(parts, axis=1)
    return full[:, :n, :]


def kernel(x, edge_index, W1, b1, W2, b2):
    n = x.shape[0]
    e = edge_index.shape[1]
    rb = 2000

    src = edge_index[0].astype(jnp.int32)
    dst = edge_index[1].astype(jnp.int32)

    grp = NW * 8 * CHUNK
    total_chunks = ((e + grp - 1) // grp) * (NW * 8)
    e_pad = total_chunks * CHUNK
    src_p = jnp.concatenate(
        [src, jnp.zeros((e_pad - e,), jnp.int32)]).reshape(total_chunks, CHUNK)
    dst_p = jnp.concatenate(
        [dst, jnp.full((e_pad - e,), n, jnp.int32)]).reshape(total_chunks, CHUNK)

    cpt = total_chunks // NW
    cap = cpt * CHUNK + 2 * CHUNK

    bsrc, bdst, nch = _bucket_kernel(total_chunks, cap)(src_p, dst_p)

    ones_tab = jnp.ones((n, 16), jnp.float32)
    degacc = _scatter_kernel(cap, 16, 1)(bsrc, bdst, nch, ones_tab, ones_tab)
    dg = _reassemble(degacc, n)
    d0 = dg[0][:, 0:1]
    d1 = dg[1][:, 0:1]

    h1p = _tc1(x, W1, d0, d1, rb)
    ta = h1p[:, :32]
    tb = h1p[:, 32:]

    acc1 = _scatter_kernel(cap, 32, 2)(bsrc, bdst, nch, ta, tb)
    a = _reassemble(acc1, n)
    h2p = _tc2(a[0], a[1], h1p, d0, d1, W2, b1.reshape(1, -1), rb)

    acc2 = _scatter_kernel(cap, 16, 1)(bsrc, bdst, nch, h2p, h2p)
    p = _reassemble(acc2, n)
    return _tc3(p[0], p[1], h2p, d0, d1, b2.reshape(1, -1), rb)


# speedup vs baseline: 11.8065x; 1.2066x over previous
import functools

import jax
import jax.numpy as jnp
from jax import lax
from jax.experimental import pallas as pl
from jax.experimental.pallas import tpu as pltpu
from jax.experimental.pallas import tpu_sc as plsc

NC = 2
NS = 16
NW = NC * NS
CHUNK = 128
R = 5
RANGE = 10016
ACC_ROWS = 10048
STRIPE = ACC_ROWS // NS
TRASH = RANGE
DEG_ROWS = 51200
DEG_STRIPE = DEG_ROWS // NS


def _mesh():
    return plsc.VectorSubcoreMesh(core_axis_name="c", subcore_axis_name="s")


def _sc_params():
    return pltpu.CompilerParams(use_tc_tiling_on_sc=False,
                                needs_layout_passes=False)


def _fill_const(buf, rows, width, value):
    def body(i, _):
        for w0 in range(width // 16):
            buf[i, pl.ds(w0 * 16, 16)] = jnp.full((16,), value, jnp.float32)
        return 0
    lax.fori_loop(0, rows, body, 0)


def _bucket_kernel(total_chunks, cap):
    cpt = total_chunks // NW

    @functools.partial(
        pl.kernel,
        out_type=[
            jax.ShapeDtypeStruct((NW, R, cap), jnp.int32),
            jax.ShapeDtypeStruct((NW, R, cap), jnp.int32),
            jax.ShapeDtypeStruct((NW, 8, 16), jnp.int32),
        ],
        mesh=_mesh(),
        compiler_params=_sc_params(),
        scratch_types=[
            pltpu.VMEM((cpt, CHUNK), jnp.int32),
            pltpu.VMEM((cpt, CHUNK), jnp.int32),
            pltpu.VMEM((cap,), jnp.int32),
            pltpu.VMEM((cap,), jnp.int32),
            pltpu.VMEM((8, 16), jnp.int32),
        ],
    )
    def k(src_hbm, dst_hbm,
          bsrc_hbm, bdst_hbm, nch_hbm,
          si_v, di_v, cs_v, cd_v, cnts_v):
        c = lax.axis_index("c")
        s = lax.axis_index("s")
        g = c * NS + s
        pltpu.sync_copy(src_hbm.at[pl.ds(g * cpt, cpt)], si_v)
        pltpu.sync_copy(dst_hbm.at[pl.ds(g * cpt, cpt)], di_v)

        for r in range(R):
            lo = r * RANGE

            def scan_body(it, cnt):
                j = it // 8
                q = it % 8
                vd = di_v[j, pl.ds(q * 16, 16)]
                vs = si_v[j, pl.ds(q * 16, 16)]
                mask = (vd >= lo) & (vd < lo + RANGE)
                loc = vd - lo
                pos = plsc.cumsum(mask.astype(jnp.int32))
                idx = pos - 1 + cnt
                plsc.store_scatter(cs_v, [idx], vs, mask=mask)
                plsc.store_scatter(cd_v, [idx], loc, mask=mask)
                return cnt + pos[15]
            cnt = lax.fori_loop(0, cpt * 8, scan_body, jnp.int32(0))

            for kk in range(17):
                cs_v[pl.ds(cnt + kk * 16, 16)] = jnp.zeros((16,), jnp.int32)
                cd_v[pl.ds(cnt + kk * 16, 16)] = jnp.full((16,), TRASH,
                                                          jnp.int32)
            nc = 2 * ((cnt + 2 * CHUNK - 1) // (2 * CHUNK))
            cnts_v[r, :] = jnp.full((16,), 1, jnp.int32) * nc
            pltpu.sync_copy(cs_v, bsrc_hbm.at[g, r])
            pltpu.sync_copy(cd_v, bdst_hbm.at[g, r])
        for r in range(R, 8):
            cnts_v[r, :] = jnp.zeros((16,), jnp.int32)
        pltpu.sync_copy(cnts_v, nch_hbm.at[g])

    return k


def _scatter_kernel(cap, width, buckets_per_tile):
    @functools.partial(
        pl.kernel,
        out_type=jax.ShapeDtypeStruct((NC, R * ACC_ROWS, width), jnp.float32),
        mesh=_mesh(),
        compiler_params=_sc_params(),
        scratch_types=[
            pltpu.VMEM((cap,), jnp.int32),
            pltpu.VMEM((cap,), jnp.int32),
            pltpu.VMEM((CHUNK,), jnp.int32),
            pltpu.VMEM((CHUNK, width), jnp.float32),
            pltpu.VMEM((CHUNK, width), jnp.float32),
            pltpu.VMEM((16,), jnp.int32),
            pltpu.VMEM((STRIPE, width), jnp.float32),
            pltpu.VMEM_SHARED((ACC_ROWS, width), jnp.float32),
            pltpu.SemaphoreType.DMA,
            pltpu.SemaphoreType.DMA,
        ],
    )
    def k(bsrc_hbm, bdst_hbm, nch_hbm, ta_hbm, tb_hbm, out_hbm,
          sb_v, db_v, dchunk_v, gbuf0_v, gbuf1_v, ncv_v, zeros_v, acc_s,
          sem0, sem1):
        c = lax.axis_index("c")
        s = lax.axis_index("s")
        _fill_const(zeros_v, STRIPE, width, 0.0)
        pltpu.sync_copy(zeros_v, acc_s.at[pl.ds(s * STRIPE, STRIPE)])
        plsc.subcore_barrier()

        def run_round(table, r):
            for kk in range(buckets_per_tile):
                if buckets_per_tile == 2:
                    bt = 2 * s + kk
                else:
                    bt = c * NS + s
                pltpu.sync_copy(nch_hbm.at[bt, r], ncv_v)
                nc = ncv_v[...][0]
                pltpu.sync_copy(bsrc_hbm.at[bt, r], sb_v)
                pltpu.sync_copy(bdst_hbm.at[bt, r], db_v)

                bufs = (gbuf0_v, gbuf1_v)
                sems = (sem0, sem1)

                def gcopy(j, b):
                    return pltpu.make_async_copy(
                        table.at[sb_v.at[pl.ds(j * CHUNK, CHUNK)]],
                        bufs[b], sems[b])

                @pl.when(nc >= 2)
                def _():
                    gcopy(0, 0).start()
                    gcopy(1, 1).start()

                def pair_body(m, _):
                    for b in range(2):
                        j = 2 * m + b
                        gcopy(j, b).wait()
                        for q in range(CHUNK // 16):
                            dchunk_v[pl.ds(q * 16, 16)] = (
                                db_v[pl.ds(j * CHUNK + q * 16, 16)])
                        pltpu.sync_copy(bufs[b], acc_s.at[dchunk_v], add=True)

                        @pl.when(j + 2 < nc)
                        def _():
                            gcopy(j + 2, b).start()
                    return 0
                lax.fori_loop(0, nc // 2, pair_body, 0)

        for r in range(R):
            if buckets_per_tile == 2:
                @pl.when(c == 0)
                def _():
                    run_round(ta_hbm, r)

                @pl.when(c == 1)
                def _():
                    run_round(tb_hbm, r)
            else:
                run_round(ta_hbm, r)
            plsc.subcore_barrier()
            pltpu.sync_copy(
                acc_s.at[pl.ds(s * STRIPE, STRIPE)],
                out_hbm.at[c, pl.ds(r * ACC_ROWS + s * STRIPE, STRIPE)])
            if r < R - 1:
                pltpu.sync_copy(zeros_v,
                                acc_s.at[pl.ds(s * STRIPE, STRIPE)])
            plsc.subcore_barrier()

    return k


def _tc1(x, W1, d0, d1, rb):
    n, d_in = x.shape
    d_out = W1.shape[1]

    def body(x_ref, w_ref, d0_ref, d1_ref, o_ref):
        dinv = lax.rsqrt(d0_ref[...] + d1_ref[...] + 1.0)
        h = jnp.dot(x_ref[...], w_ref[...], preferred_element_type=jnp.float32)
        o_ref[...] = h * dinv

    return pl.pallas_call(
        body,
        grid=(n // rb,),
        in_specs=[
            pl.BlockSpec((rb, d_in), lambda i: (i, 0)),
            pl.BlockSpec((d_in, d_out), lambda i: (0, 0)),
            pl.BlockSpec((rb, 1), lambda i: (i, 0)),
            pl.BlockSpec((rb, 1), lambda i: (i, 0)),
        ],
        out_specs=pl.BlockSpec((rb, d_out), lambda i: (i, 0)),
        out_shape=jax.ShapeDtypeStruct((n, d_out), jnp.float32),
    )(x, W1, d0, d1)


def _tc2(a0, a1, h1p, d0, d1, W2, b1, rb):
    n, half = a0.shape
    d2 = W2.shape[1]

    def body(a0_ref, a1_ref, h_ref, d0_ref, d1_ref, w_ref, b_ref, o_ref):
        dinv = lax.rsqrt(d0_ref[...] + d1_ref[...] + 1.0)
        u = jnp.concatenate([a0_ref[...], a1_ref[...]], axis=1) + h_ref[...]
        z = jnp.maximum(u * dinv + b_ref[...], 0.0)
        h2 = jnp.dot(z, w_ref[...], preferred_element_type=jnp.float32)
        o_ref[...] = h2 * dinv

    return pl.pallas_call(
        body,
        grid=(n // rb,),
        in_specs=[
            pl.BlockSpec((rb, half), lambda i: (i, 0)),
            pl.BlockSpec((rb, half), lambda i: (i, 0)),
            pl.BlockSpec((rb, 2 * half), lambda i: (i, 0)),
            pl.BlockSpec((rb, 1), lambda i: (i, 0)),
            pl.BlockSpec((rb, 1), lambda i: (i, 0)),
            pl.BlockSpec((2 * half, d2), lambda i: (0, 0)),
            pl.BlockSpec((1, 2 * half), lambda i: (0, 0)),
        ],
        out_specs=pl.BlockSpec((rb, d2), lambda i: (i, 0)),
        out_shape=jax.ShapeDtypeStruct((n, d2), jnp.float32),
    )(a0, a1, h1p, d0, d1, W2, b1)


def _tc3(p0, p1, h2p, d0, d1, b2, rb):
    n, d2 = h2p.shape

    def body(p0_ref, p1_ref, h_ref, d0_ref, d1_ref, b_ref, o_ref):
        dinv = lax.rsqrt(d0_ref[...] + d1_ref[...] + 1.0)
        u = p0_ref[...] + p1_ref[...] + h_ref[...]
        o_ref[...] = jnp.maximum(u * dinv + b_ref[...], 0.0)

    return pl.pallas_call(
        body,
        grid=(n // rb,),
        in_specs=[
            pl.BlockSpec((rb, d2), lambda i: (i, 0)),
            pl.BlockSpec((rb, d2), lambda i: (i, 0)),
            pl.BlockSpec((rb, d2), lambda i: (i, 0)),
            pl.BlockSpec((rb, 1), lambda i: (i, 0)),
            pl.BlockSpec((rb, 1), lambda i: (i, 0)),
            pl.BlockSpec((1, d2), lambda i: (0, 0)),
        ],
        out_specs=pl.BlockSpec((rb, d2), lambda i: (i, 0)),
        out_shape=jax.ShapeDtypeStruct((n, d2), jnp.float32),
    )(p0, p1, h2p, d0, d1, b2)


def _reassemble(y, n):
    parts = [y[:, r * ACC_ROWS: r * ACC_ROWS + RANGE, :] for r in range(R)]
    full = jnp.concatenate(parts, axis=1)
    return full[:, :n, :]


def kernel(x, edge_index, W1, b1, W2, b2):
    n = x.shape[0]
    e = edge_index.shape[1]
    rb = 2000

    src = edge_index[0].astype(jnp.int32)
    dst = edge_index[1].astype(jnp.int32)

    grp = NW * 8 * CHUNK
    total_chunks = ((e + grp - 1) // grp) * (NW * 8)
    e_pad = total_chunks * CHUNK
    src_p = jnp.concatenate(
        [src, jnp.zeros((e_pad - e,), jnp.int32)]).reshape(total_chunks, CHUNK)
    dst_p = jnp.concatenate(
        [dst, jnp.full((e_pad - e,), n, jnp.int32)]).reshape(total_chunks, CHUNK)

    cpt = total_chunks // NW
    cap = cpt * CHUNK + 3 * CHUNK

    bsrc, bdst, nch = _bucket_kernel(total_chunks, cap)(src_p, dst_p)

    ones_tab = jnp.ones((n, 16), jnp.float32)
    degacc = _scatter_kernel(cap, 16, 1)(bsrc, bdst, nch, ones_tab, ones_tab)
    dg = _reassemble(degacc, n)
    d0 = dg[0][:, 0:1]
    d1 = dg[1][:, 0:1]

    h1p = _tc1(x, W1, d0, d1, rb)
    ta = h1p[:, :32]
    tb = h1p[:, 32:]

    acc1 = _scatter_kernel(cap, 32, 2)(bsrc, bdst, nch, ta, tb)
    a = _reassemble(acc1, n)
    h2p = _tc2(a[0], a[1], h1p, d0, d1, W2, b1.reshape(1, -1), rb)

    acc2 = _scatter_kernel(cap, 16, 1)(bsrc, bdst, nch, h2p, h2p)
    p = _reassemble(acc2, n)
    return _tc3(p[0], p[1], h2p, d0, d1, b2.reshape(1, -1), rb)


# speedup vs baseline: 12.2238x; 1.0353x over previous
import functools

import jax
import jax.numpy as jnp
from jax import lax
from jax.experimental import pallas as pl
from jax.experimental.pallas import tpu as pltpu
from jax.experimental.pallas import tpu_sc as plsc

NC = 2
NS = 16
NW = NC * NS
CHUNK = 128
R = 5
RANGE = 10016
ACC_ROWS = 10048
STRIPE = ACC_ROWS // NS
TRASH = RANGE


def _mesh():
    return plsc.VectorSubcoreMesh(core_axis_name="c", subcore_axis_name="s")


def _sc_params():
    return pltpu.CompilerParams(use_tc_tiling_on_sc=False,
                                needs_layout_passes=False)


def _fill_const(buf, rows, width, value):
    def body(i, _):
        for w0 in range(width // 16):
            buf[i, pl.ds(w0 * 16, 16)] = jnp.full((16,), value, jnp.float32)
        return 0
    lax.fori_loop(0, rows, body, 0)


def _bucket_kernel(total_chunks, capc):
    cpt = total_chunks // NW

    @functools.partial(
        pl.kernel,
        out_type=[
            jax.ShapeDtypeStruct((NW, R, capc, CHUNK), jnp.int32),
            jax.ShapeDtypeStruct((NW, R, capc, CHUNK), jnp.int32),
            jax.ShapeDtypeStruct((NW, 8, 16), jnp.int32),
            jax.ShapeDtypeStruct((NC, R * ACC_ROWS, 16), jnp.float32),
        ],
        mesh=_mesh(),
        compiler_params=_sc_params(),
        scratch_types=[
            pltpu.VMEM((cpt, CHUNK), jnp.int32),
            pltpu.VMEM((cpt, CHUNK), jnp.int32),
            pltpu.VMEM((capc, CHUNK), jnp.int32),
            pltpu.VMEM((capc, CHUNK), jnp.int32),
            pltpu.VMEM((8, 16), jnp.int32),
            pltpu.VMEM((CHUNK, 16), jnp.float32),
            pltpu.VMEM((STRIPE, 16), jnp.float32),
            pltpu.VMEM_SHARED((ACC_ROWS, 16), jnp.float32),
        ],
    )
    def k(src_hbm, dst_hbm, bsrc_hbm, bdst_hbm, nch_hbm, deg_hbm,
          si_v, di_v, cs_v, cd_v, cnts_v, ones_v, zeros_v, acc_s):
        c = lax.axis_index("c")
        s = lax.axis_index("s")
        g = c * NS + s
        _fill_const(ones_v, CHUNK, 16, 1.0)
        _fill_const(zeros_v, STRIPE, 16, 0.0)
        pltpu.sync_copy(zeros_v, acc_s.at[pl.ds(s * STRIPE, STRIPE)])
        pltpu.sync_copy(src_hbm.at[pl.ds(g * cpt, cpt)], si_v)
        pltpu.sync_copy(dst_hbm.at[pl.ds(g * cpt, cpt)], di_v)
        plsc.subcore_barrier()

        for r in range(R):
            lo = r * RANGE

            def scan_body(it, cnt):
                j = it // 8
                q = it % 8
                vd = di_v[j, pl.ds(q * 16, 16)]
                vs = si_v[j, pl.ds(q * 16, 16)]
                mask = (vd >= lo) & (vd < lo + RANGE)
                loc = vd - lo
                pos = plsc.cumsum(mask.astype(jnp.int32))
                idx = pos - 1 + cnt
                row = lax.shift_right_logical(idx, 7)
                col = lax.bitwise_and(idx, 127)
                plsc.store_scatter(cs_v, [row, col], vs, mask=mask)
                plsc.store_scatter(cd_v, [row, col], loc, mask=mask)
                return cnt + pos[15]
            cnt = lax.fori_loop(0, cpt * 8, scan_body, jnp.int32(0))

            base = lax.iota(jnp.int32, 16)
            for kk in range(33):
                idx = cnt + kk * 16 + base
                row = lax.shift_right_logical(idx, 7)
                col = lax.bitwise_and(idx, 127)
                plsc.store_scatter(cs_v, [row, col],
                                   jnp.zeros((16,), jnp.int32))
                plsc.store_scatter(cd_v, [row, col],
                                   jnp.full((16,), TRASH, jnp.int32))
            nc = 4 * ((cnt + 4 * CHUNK - 1) // (4 * CHUNK))
            cnts_v[r, :] = jnp.full((16,), 1, jnp.int32) * nc
            pltpu.sync_copy(cs_v, bsrc_hbm.at[g, r])
            pltpu.sync_copy(cd_v, bdst_hbm.at[g, r])

            def deg_body(j, _):
                pltpu.sync_copy(ones_v, acc_s.at[cd_v.at[j]], add=True)
                return 0
            lax.fori_loop(0, nc, deg_body, 0)
            plsc.subcore_barrier()
            pltpu.sync_copy(
                acc_s.at[pl.ds(s * STRIPE, STRIPE)],
                deg_hbm.at[c, pl.ds(r * ACC_ROWS + s * STRIPE, STRIPE)])
            if r < R - 1:
                pltpu.sync_copy(zeros_v, acc_s.at[pl.ds(s * STRIPE, STRIPE)])
            plsc.subcore_barrier()

        for r in range(R, 8):
            cnts_v[r, :] = jnp.zeros((16,), jnp.int32)
        pltpu.sync_copy(cnts_v, nch_hbm.at[g])

    return k


def _scatter_kernel(capc, width, buckets_per_tile):
    @functools.partial(
        pl.kernel,
        out_type=jax.ShapeDtypeStruct((NC, R * ACC_ROWS, width), jnp.float32),
        mesh=_mesh(),
        compiler_params=_sc_params(),
        scratch_types=[
            pltpu.VMEM((capc, CHUNK), jnp.int32),
            pltpu.VMEM((capc, CHUNK), jnp.int32),
            pltpu.VMEM((CHUNK, width), jnp.float32),
            pltpu.VMEM((CHUNK, width), jnp.float32),
            pltpu.VMEM((CHUNK, width), jnp.float32),
            pltpu.VMEM((CHUNK, width), jnp.float32),
            pltpu.VMEM((16,), jnp.int32),
            pltpu.VMEM((STRIPE, width), jnp.float32),
            pltpu.VMEM_SHARED((ACC_ROWS, width), jnp.float32),
            pltpu.SemaphoreType.DMA,
            pltpu.SemaphoreType.DMA,
            pltpu.SemaphoreType.DMA,
            pltpu.SemaphoreType.DMA,
        ],
    )
    def k(bsrc_hbm, bdst_hbm, nch_hbm, ta_hbm, tb_hbm, out_hbm,
          sb_v, db_v, gbuf0_v, gbuf1_v, gbuf2_v, gbuf3_v, ncv_v, zeros_v,
          acc_s, sem0, sem1, sem2, sem3):
        c = lax.axis_index("c")
        s = lax.axis_index("s")
        _fill_const(zeros_v, STRIPE, width, 0.0)
        pltpu.sync_copy(zeros_v, acc_s.at[pl.ds(s * STRIPE, STRIPE)])
        plsc.subcore_barrier()

        def run_round(table, r):
            for kk in range(buckets_per_tile):
                if buckets_per_tile == 2:
                    bt = 2 * s + kk
                else:
                    bt = c * NS + s
                pltpu.sync_copy(nch_hbm.at[bt, r], ncv_v)
                nc = ncv_v[...][0]
                pltpu.sync_copy(bsrc_hbm.at[bt, r], sb_v)
                pltpu.sync_copy(bdst_hbm.at[bt, r], db_v)

                bufs = (gbuf0_v, gbuf1_v, gbuf2_v, gbuf3_v)
                sems = (sem0, sem1, sem2, sem3)

                def gcopy(j, b):
                    return pltpu.make_async_copy(
                        table.at[sb_v.at[j]], bufs[b], sems[b])

                @pl.when(nc >= 4)
                def _():
                    for b in range(4):
                        gcopy(b, b).start()

                def quad_body(m, _):
                    for b in range(4):
                        j = 4 * m + b
                        gcopy(j, b).wait()
                        pltpu.sync_copy(bufs[b], acc_s.at[db_v.at[j]],
                                        add=True)

                        @pl.when(j + 4 < nc)
                        def _():
                            gcopy(j + 4, b).start()
                    return 0
                lax.fori_loop(0, nc // 4, quad_body, 0)

        for r in range(R):
            if buckets_per_tile == 2:
                @pl.when(c == 0)
                def _():
                    run_round(ta_hbm, r)

                @pl.when(c == 1)
                def _():
                    run_round(tb_hbm, r)
            else:
                run_round(ta_hbm, r)
            plsc.subcore_barrier()
            pltpu.sync_copy(
                acc_s.at[pl.ds(s * STRIPE, STRIPE)],
                out_hbm.at[c, pl.ds(r * ACC_ROWS + s * STRIPE, STRIPE)])
            if r < R - 1:
                pltpu.sync_copy(zeros_v,
                                acc_s.at[pl.ds(s * STRIPE, STRIPE)])
            plsc.subcore_barrier()

    return k


def _tc1(x, W1, d0, d1, rb):
    n, d_in = x.shape
    d_out = W1.shape[1]

    def body(x_ref, w_ref, d0_ref, d1_ref, o_ref):
        dinv = lax.rsqrt(d0_ref[...] + d1_ref[...] + 1.0)
        h = jnp.dot(x_ref[...], w_ref[...], preferred_element_type=jnp.float32)
        o_ref[...] = h * dinv

    return pl.pallas_call(
        body,
        grid=(n // rb,),
        in_specs=[
            pl.BlockSpec((rb, d_in), lambda i: (i, 0)),
            pl.BlockSpec((d_in, d_out), lambda i: (0, 0)),
            pl.BlockSpec((rb, 1), lambda i: (i, 0)),
            pl.BlockSpec((rb, 1), lambda i: (i, 0)),
        ],
        out_specs=pl.BlockSpec((rb, d_out), lambda i: (i, 0)),
        out_shape=jax.ShapeDtypeStruct((n, d_out), jnp.float32),
    )(x, W1, d0, d1)


def _tc2(a0, a1, h1p, d0, d1, W2, b1, rb):
    n, half = a0.shape
    d2 = W2.shape[1]

    def body(a0_ref, a1_ref, h_ref, d0_ref, d1_ref, w_ref, b_ref, o_ref):
        dinv = lax.rsqrt(d0_ref[...] + d1_ref[...] + 1.0)
        u = jnp.concatenate([a0_ref[...], a1_ref[...]], axis=1) + h_ref[...]
        z = jnp.maximum(u * dinv + b_ref[...], 0.0)
        h2 = jnp.dot(z, w_ref[...], preferred_element_type=jnp.float32)
        o_ref[...] = h2 * dinv

    return pl.pallas_call(
        body,
        grid=(n // rb,),
        in_specs=[
            pl.BlockSpec((rb, half), lambda i: (i, 0)),
            pl.BlockSpec((rb, half), lambda i: (i, 0)),
            pl.BlockSpec((rb, 2 * half), lambda i: (i, 0)),
            pl.BlockSpec((rb, 1), lambda i: (i, 0)),
            pl.BlockSpec((rb, 1), lambda i: (i, 0)),
            pl.BlockSpec((2 * half, d2), lambda i: (0, 0)),
            pl.BlockSpec((1, 2 * half), lambda i: (0, 0)),
        ],
        out_specs=pl.BlockSpec((rb, d2), lambda i: (i, 0)),
        out_shape=jax.ShapeDtypeStruct((n, d2), jnp.float32),
    )(a0, a1, h1p, d0, d1, W2, b1)


def _tc3(p0, p1, h2p, d0, d1, b2, rb):
    n, d2 = h2p.shape

    def body(p0_ref, p1_ref, h_ref, d0_ref, d1_ref, b_ref, o_ref):
        dinv = lax.rsqrt(d0_ref[...] + d1_ref[...] + 1.0)
        u = p0_ref[...] + p1_ref[...] + h_ref[...]
        o_ref[...] = jnp.maximum(u * dinv + b_ref[...], 0.0)

    return pl.pallas_call(
        body,
        grid=(n // rb,),
        in_specs=[
            pl.BlockSpec((rb, d2), lambda i: (i, 0)),
            pl.BlockSpec((rb, d2), lambda i: (i, 0)),
            pl.BlockSpec((rb, d2), lambda i: (i, 0)),
            pl.BlockSpec((rb, 1), lambda i: (i, 0)),
            pl.BlockSpec((rb, 1), lambda i: (i, 0)),
            pl.BlockSpec((1, d2), lambda i: (0, 0)),
        ],
        out_specs=pl.BlockSpec((rb, d2), lambda i: (i, 0)),
        out_shape=jax.ShapeDtypeStruct((n, d2), jnp.float32),
    )(p0, p1, h2p, d0, d1, b2)


def _reassemble(y, n):
    parts = [y[:, r * ACC_ROWS: r * ACC_ROWS + RANGE, :] for r in range(R)]
    full = jnp.concatenate(parts, axis=1)
    return full[:, :n, :]


def kernel(x, edge_index, W1, b1, W2, b2):
    n = x.shape[0]
    e = edge_index.shape[1]
    rb = 2000

    src = edge_index[0].astype(jnp.int32)
    dst = edge_index[1].astype(jnp.int32)

    grp = NW * 8 * CHUNK
    total_chunks = ((e + grp - 1) // grp) * (NW * 8)
    e_pad = total_chunks * CHUNK
    src_p = jnp.concatenate(
        [src, jnp.zeros((e_pad - e,), jnp.int32)]).reshape(total_chunks, CHUNK)
    dst_p = jnp.concatenate(
        [dst, jnp.full((e_pad - e,), n, jnp.int32)]).reshape(total_chunks, CHUNK)

    cpt = total_chunks // NW
    capc = cpt + 5

    bsrc, bdst, nch, degacc = _bucket_kernel(total_chunks, capc)(src_p, dst_p)
    dg = _reassemble(degacc, n)
    d0 = dg[0][:, 0:1]
    d1 = dg[1][:, 0:1]

    h1p = _tc1(x, W1, d0, d1, rb)
    ta = h1p[:, :32]
    tb = h1p[:, 32:]

    acc1 = _scatter_kernel(capc, 32, 2)(bsrc, bdst, nch, ta, tb)
    a = _reassemble(acc1, n)
    h2p = _tc2(a[0], a[1], h1p, d0, d1, W2, b1.reshape(1, -1), rb)

    acc2 = _scatter_kernel(capc, 16, 1)(bsrc, bdst, nch, h2p, h2p)
    p = _reassemble(acc2, n)
    return _tc3(p[0], p[1], h2p, d0, d1, b2.reshape(1, -1), rb)


# speedup vs baseline: 12.3671x; 1.0117x over previous
import functools

import jax
import jax.numpy as jnp
from jax import lax
from jax.experimental import pallas as pl
from jax.experimental.pallas import tpu as pltpu
from jax.experimental.pallas import tpu_sc as plsc

NC = 2
NS = 16
NW = NC * NS
CHUNK = 128
BLK = 40
PAGE = 512
R = 5
RANGE = 10016
ACC_ROWS = 10048
STRIPE = ACC_ROWS // NS
TRASH = RANGE


def _mesh():
    return plsc.VectorSubcoreMesh(core_axis_name="c", subcore_axis_name="s")


def _sc_params():
    return pltpu.CompilerParams(use_tc_tiling_on_sc=False,
                                needs_layout_passes=False)


def _fill_const(buf, rows, width, value):
    def body(i, _):
        for w0 in range(width // 16):
            buf[i, pl.ds(w0 * 16, 16)] = jnp.full((16,), value, jnp.float32)
        return 0
    lax.fori_loop(0, rows, body, 0)


def _bucket_kernel(total_chunks, capp):
    cpt = total_chunks // NW

    @functools.partial(
        pl.kernel,
        out_type=[
            jax.ShapeDtypeStruct((NW, R, capp, PAGE), jnp.int32),
            jax.ShapeDtypeStruct((NW, R, capp, PAGE), jnp.int32),
            jax.ShapeDtypeStruct((NW, 8, 16), jnp.int32),
            jax.ShapeDtypeStruct((NC, R * ACC_ROWS, 16), jnp.float32),
        ],
        mesh=_mesh(),
        compiler_params=_sc_params(),
        scratch_types=[
            pltpu.VMEM((BLK, CHUNK), jnp.int32),
            pltpu.VMEM((BLK, CHUNK), jnp.int32),
            pltpu.VMEM((capp, PAGE), jnp.int32),
            pltpu.VMEM((capp, PAGE), jnp.int32),
            pltpu.VMEM((8, 16), jnp.int32),
            pltpu.VMEM((PAGE, 16), jnp.float32),
            pltpu.VMEM((STRIPE, 16), jnp.float32),
            pltpu.VMEM_SHARED((ACC_ROWS, 16), jnp.float32),
        ],
    )
    def k(src_hbm, dst_hbm, bsrc_hbm, bdst_hbm, nch_hbm, deg_hbm,
          si_v, di_v, cs_v, cd_v, cnts_v, ones_v, zeros_v, acc_s):
        c = lax.axis_index("c")
        s = lax.axis_index("s")
        g = c * NS + s
        _fill_const(ones_v, PAGE, 16, 1.0)
        _fill_const(zeros_v, STRIPE, 16, 0.0)
        pltpu.sync_copy(zeros_v, acc_s.at[pl.ds(s * STRIPE, STRIPE)])
        plsc.subcore_barrier()

        for r in range(R):
            lo = r * RANGE

            def blk_body(bi, cnt):
                pltpu.sync_copy(
                    src_hbm.at[pl.ds(g * cpt + bi * BLK, BLK)], si_v)
                pltpu.sync_copy(
                    dst_hbm.at[pl.ds(g * cpt + bi * BLK, BLK)], di_v)

                def scan_body(it, cnt):
                    j = it // 8
                    q = it % 8
                    vd = di_v[j, pl.ds(q * 16, 16)]
                    vs = si_v[j, pl.ds(q * 16, 16)]
                    mask = (vd >= lo) & (vd < lo + RANGE)
                    loc = vd - lo
                    pos = plsc.cumsum(mask.astype(jnp.int32))
                    idx = pos - 1 + cnt
                    row = lax.shift_right_logical(idx, 9)
                    col = lax.bitwise_and(idx, 511)
                    plsc.store_scatter(cs_v, [row, col], vs, mask=mask)
                    plsc.store_scatter(cd_v, [row, col], loc, mask=mask)
                    return cnt + pos[15]
                return lax.fori_loop(0, BLK * 8, scan_body, cnt)
            cnt = lax.fori_loop(0, cpt // BLK, blk_body, jnp.int32(0))

            base = lax.iota(jnp.int32, 16)
            for kk in range(33):
                idx = cnt + kk * 16 + base
                row = lax.shift_right_logical(idx, 9)
                col = lax.bitwise_and(idx, 511)
                plsc.store_scatter(cs_v, [row, col],
                                   jnp.zeros((16,), jnp.int32))
                plsc.store_scatter(cd_v, [row, col],
                                   jnp.full((16,), TRASH, jnp.int32))
            nc = (cnt + PAGE - 1) // PAGE
            cnts_v[r, :] = jnp.full((16,), 1, jnp.int32) * nc
            pltpu.sync_copy(cs_v, bsrc_hbm.at[g, r])
            pltpu.sync_copy(cd_v, bdst_hbm.at[g, r])

            def deg_body(j, _):
                pltpu.sync_copy(ones_v, acc_s.at[cd_v.at[j]], add=True)
                return 0
            lax.fori_loop(0, nc, deg_body, 0)
            plsc.subcore_barrier()
            pltpu.sync_copy(
                acc_s.at[pl.ds(s * STRIPE, STRIPE)],
                deg_hbm.at[c, pl.ds(r * ACC_ROWS + s * STRIPE, STRIPE)])
            if r < R - 1:
                pltpu.sync_copy(zeros_v, acc_s.at[pl.ds(s * STRIPE, STRIPE)])
            plsc.subcore_barrier()

        for r in range(R, 8):
            cnts_v[r, :] = jnp.zeros((16,), jnp.int32)
        pltpu.sync_copy(cnts_v, nch_hbm.at[g])

    return k


def _scatter_kernel(capp, width, buckets_per_tile):
    @functools.partial(
        pl.kernel,
        out_type=jax.ShapeDtypeStruct((NC, R * ACC_ROWS, width), jnp.float32),
        mesh=_mesh(),
        compiler_params=_sc_params(),
        scratch_types=[
            pltpu.VMEM((capp, PAGE), jnp.int32),
            pltpu.VMEM((capp, PAGE), jnp.int32),
            pltpu.VMEM((PAGE, width), jnp.float32),
            pltpu.VMEM((PAGE, width), jnp.float32),
            pltpu.VMEM((16,), jnp.int32),
            pltpu.VMEM((STRIPE, width), jnp.float32),
            pltpu.VMEM_SHARED((ACC_ROWS, width), jnp.float32),
            pltpu.SemaphoreType.DMA,
            pltpu.SemaphoreType.DMA,
        ],
    )
    def k(bsrc_hbm, bdst_hbm, nch_hbm, ta_hbm, tb_hbm, out_hbm,
          sb_v, db_v, gbuf0_v, gbuf1_v, ncv_v, zeros_v,
          acc_s, sem0, sem1):
        c = lax.axis_index("c")
        s = lax.axis_index("s")
        _fill_const(zeros_v, STRIPE, width, 0.0)
        pltpu.sync_copy(zeros_v, acc_s.at[pl.ds(s * STRIPE, STRIPE)])
        plsc.subcore_barrier()

        def run_round(table, r):
            for kk in range(buckets_per_tile):
                if buckets_per_tile == 2:
                    bt = 2 * s + kk
                else:
                    bt = c * NS + s
                pltpu.sync_copy(nch_hbm.at[bt, r], ncv_v)
                nc = ncv_v[...][0]
                pltpu.sync_copy(bsrc_hbm.at[bt, r], sb_v)
                pltpu.sync_copy(bdst_hbm.at[bt, r], db_v)

                bufs = (gbuf0_v, gbuf1_v)
                sems = (sem0, sem1)

                def gcopy(j, b):
                    return pltpu.make_async_copy(
                        table.at[sb_v.at[j]], bufs[b], sems[b])

                @pl.when(nc >= 1)
                def _():
                    gcopy(0, 0).start()

                @pl.when(nc >= 2)
                def _():
                    gcopy(1, 1).start()

                def pair_body(m, _):
                    for b in range(2):
                        j = 2 * m + b

                        @pl.when(j < nc)
                        def _():
                            gcopy(j, b).wait()
                            pltpu.sync_copy(bufs[b], acc_s.at[db_v.at[j]],
                                            add=True)

                            @pl.when(j + 2 < nc)
                            def _():
                                gcopy(j + 2, b).start()
                    return 0
                lax.fori_loop(0, (nc + 1) // 2, pair_body, 0)

        for r in range(R):
            if buckets_per_tile == 2:
                @pl.when(c == 0)
                def _():
                    run_round(ta_hbm, r)

                @pl.when(c == 1)
                def _():
                    run_round(tb_hbm, r)
            else:
                run_round(ta_hbm, r)
            plsc.subcore_barrier()
            pltpu.sync_copy(
                acc_s.at[pl.ds(s * STRIPE, STRIPE)],
                out_hbm.at[c, pl.ds(r * ACC_ROWS + s * STRIPE, STRIPE)])
            if r < R - 1:
                pltpu.sync_copy(zeros_v,
                                acc_s.at[pl.ds(s * STRIPE, STRIPE)])
            plsc.subcore_barrier()

    return k


def _tc1(x, W1, d0, d1, rb):
    n, d_in = x.shape
    d_out = W1.shape[1]

    def body(x_ref, w_ref, d0_ref, d1_ref, o_ref):
        dinv = lax.rsqrt(d0_ref[...] + d1_ref[...] + 1.0)
        h = jnp.dot(x_ref[...], w_ref[...], preferred_element_type=jnp.float32)
        o_ref[...] = h * dinv

    return pl.pallas_call(
        body,
        grid=(n // rb,),
        in_specs=[
            pl.BlockSpec((rb, d_in), lambda i: (i, 0)),
            pl.BlockSpec((d_in, d_out), lambda i: (0, 0)),
            pl.BlockSpec((rb, 1), lambda i: (i, 0)),
            pl.BlockSpec((rb, 1), lambda i: (i, 0)),
        ],
        out_specs=pl.BlockSpec((rb, d_out), lambda i: (i, 0)),
        out_shape=jax.ShapeDtypeStruct((n, d_out), jnp.float32),
    )(x, W1, d0, d1)


def _tc2(a0, a1, h1p, d0, d1, W2, b1, rb):
    n, half = a0.shape
    d2 = W2.shape[1]

    def body(a0_ref, a1_ref, h_ref, d0_ref, d1_ref, w_ref, b_ref, o_ref):
        dinv = lax.rsqrt(d0_ref[...] + d1_ref[...] + 1.0)
        u = jnp.concatenate([a0_ref[...], a1_ref[...]], axis=1) + h_ref[...]
        z = jnp.maximum(u * dinv + b_ref[...], 0.0)
        h2 = jnp.dot(z, w_ref[...], preferred_element_type=jnp.float32)
        o_ref[...] = h2 * dinv

    return pl.pallas_call(
        body,
        grid=(n // rb,),
        in_specs=[
            pl.BlockSpec((rb, half), lambda i: (i, 0)),
            pl.BlockSpec((rb, half), lambda i: (i, 0)),
            pl.BlockSpec((rb, 2 * half), lambda i: (i, 0)),
            pl.BlockSpec((rb, 1), lambda i: (i, 0)),
            pl.BlockSpec((rb, 1), lambda i: (i, 0)),
            pl.BlockSpec((2 * half, d2), lambda i: (0, 0)),
            pl.BlockSpec((1, 2 * half), lambda i: (0, 0)),
        ],
        out_specs=pl.BlockSpec((rb, d2), lambda i: (i, 0)),
        out_shape=jax.ShapeDtypeStruct((n, d2), jnp.float32),
    )(a0, a1, h1p, d0, d1, W2, b1)


def _tc3(p0, p1, h2p, d0, d1, b2, rb):
    n, d2 = h2p.shape

    def body(p0_ref, p1_ref, h_ref, d0_ref, d1_ref, b_ref, o_ref):
        dinv = lax.rsqrt(d0_ref[...] + d1_ref[...] + 1.0)
        u = p0_ref[...] + p1_ref[...] + h_ref[...]
        o_ref[...] = jnp.maximum(u * dinv + b_ref[...], 0.0)

    return pl.pallas_call(
        body,
        grid=(n // rb,),
        in_specs=[
            pl.BlockSpec((rb, d2), lambda i: (i, 0)),
            pl.BlockSpec((rb, d2), lambda i: (i, 0)),
            pl.BlockSpec((rb, d2), lambda i: (i, 0)),
            pl.BlockSpec((rb, 1), lambda i: (i, 0)),
            pl.BlockSpec((rb, 1), lambda i: (i, 0)),
            pl.BlockSpec((1, d2), lambda i: (0, 0)),
        ],
        out_specs=pl.BlockSpec((rb, d2), lambda i: (i, 0)),
        out_shape=jax.ShapeDtypeStruct((n, d2), jnp.float32),
    )(p0, p1, h2p, d0, d1, b2)


def _reassemble(y, n):
    parts = [y[:, r * ACC_ROWS: r * ACC_ROWS + RANGE, :] for r in range(R)]
    full = jnp.concatenate(parts, axis=1)
    return full[:, :n, :]


def kernel(x, edge_index, W1, b1, W2, b2):
    n = x.shape[0]
    e = edge_index.shape[1]
    rb = 2000

    src = edge_index[0].astype(jnp.int32)
    dst = edge_index[1].astype(jnp.int32)

    grp = NW * 8 * CHUNK
    total_chunks = ((e + grp - 1) // grp) * (NW * 8)
    e_pad = total_chunks * CHUNK
    src_p = jnp.concatenate(
        [src, jnp.zeros((e_pad - e,), jnp.int32)]).reshape(total_chunks, CHUNK)
    dst_p = jnp.concatenate(
        [dst, jnp.full((e_pad - e,), n, jnp.int32)]).reshape(total_chunks, CHUNK)

    cpt = total_chunks // NW
    capp = (cpt * CHUNK + 2 * PAGE) // PAGE

    bsrc, bdst, nch, degacc = _bucket_kernel(total_chunks, capp)(src_p, dst_p)
    dg = _reassemble(degacc, n)
    d0 = dg[0][:, 0:1]
    d1 = dg[1][:, 0:1]

    h1p = _tc1(x, W1, d0, d1, rb)
    ta = h1p[:, :32]
    tb = h1p[:, 32:]

    acc1 = _scatter_kernel(capp, 32, 2)(bsrc, bdst, nch, ta, tb)
    a = _reassemble(acc1, n)
    h2p = _tc2(a[0], a[1], h1p, d0, d1, W2, b1.reshape(1, -1), rb)

    acc2 = _scatter_kernel(capp, 16, 1)(bsrc, bdst, nch, h2p, h2p)
    p = _reassemble(acc2, n)
    return _tc3(p[0], p[1], h2p, d0, d1, b2.reshape(1, -1), rb)
